# baseline (speedup 1.0000x reference)
import functools

import jax
import jax.numpy as jnp
from jax import lax
from jax.experimental import pallas as pl
from jax.experimental.pallas import tpu as pltpu
from jax.experimental.pallas import tpu_sc as plsc

B = 4096
S = 50
D = 64
P = 20
T = P + S

NC = 2
NS = 16
NW = NC * NS
L = 16
DT = D // 8
LB = 128
NBUF = 4


def _splat(val):
    return jnp.full((L,), val, dtype=jnp.int32)


def _soft_embed(ids_hbm, table_hbm, learned_hbm, out_hbm,
                idx_v, lrn_v, rows_v, slab_v, pslab_v, gsem, wsem, psem):
    wid = lax.axis_index("s") * NC + lax.axis_index("c")
    pltpu.sync_copy(ids_hbm.at[:, pl.ds(wid * LB, LB)], idx_v)
    pltpu.sync_copy(learned_hbm, lrn_v)

    def fire(s, p4):
        pltpu.async_copy(table_hbm.at[idx_v.at[s]], rows_v.at[p4], gsem.at[p4])

    def drain_gather(s, p4):
        pltpu.make_async_copy(
            table_hbm.at[idx_v.at[s]], rows_v.at[p4], gsem.at[p4]
        ).wait()

    def write_slab(t, p, sem):
        for dt in range(DT):
            pltpu.async_copy(
                slab_v.at[p, pl.ds(dt * 8, 8)], out_hbm.at[t, dt, wid], sem
            )

    def drain_slab(t, p, sem):
        for dt in range(DT):
            pltpu.make_async_copy(
                slab_v.at[p, pl.ds(dt * 8, 8)], out_hbm.at[t, dt, wid], sem
            ).wait()

    for s0 in range(NBUF):
        fire(s0, s0)

    iota = lax.iota(jnp.int32, L)

    def pdrain(t, p2):
        for dt in range(DT):
            pltpu.make_async_copy(
                pslab_v.at[p2, pl.ds(dt * 8, 8)], out_hbm.at[t, dt, wid],
                psem.at[p2],
            ).wait()

    def prefix_body(t, carry):
        p2 = lax.rem(t, 2)
        p2v = _splat(p2)
        tv = _splat(t)

        @pl.when(t >= 2)
        def _():
            pdrain(t - 2, p2)

        def dbody(d, c):
            v = plsc.load_gather(lrn_v, [tv, _splat(d)])
            for c8 in range(LB // L):
                pslab_v[p2, d, pl.ds(c8 * L, L)] = v
            return c

        lax.fori_loop(0, D, dbody, 0)
        for dt in range(DT):
            pltpu.async_copy(
                pslab_v.at[p2, pl.ds(dt * 8, 8)], out_hbm.at[t, dt, wid],
                psem.at[p2],
            )
        return carry

    lax.fori_loop(0, P, prefix_body, 0)

    def body(s, carry):
        p = lax.rem(s, 2)
        p4 = lax.rem(s, NBUF)
        pv = _splat(p)
        p4v = _splat(p4)
        drain_gather(s, p4)

        @pl.when(s >= 2)
        def _():
            drain_slab(P + s - 2, p, wsem.at[p])

        def kbody(k, c):
            rot = jnp.bitwise_and(iota + k, L - 1)
            for d0 in range(0, D, L):
                colv = rot + d0
                for bl0 in range(0, LB, L):
                    rowv = iota + bl0
                    v = plsc.load_gather(rows_v, [p4v, rowv, colv])
                    plsc.store_scatter(slab_v, [pv, colv, rowv], v)
            return c

        lax.fori_loop(0, L, kbody, 0)

        @pl.when(s < S - NBUF)
        def _():
            fire(s + NBUF, p4)

        write_slab(P + s, p, wsem.at[p])
        return carry

    lax.fori_loop(0, S, body, 0)
    drain_slab(P + S - 2, 0, wsem.at[0])
    drain_slab(P + S - 1, 1, wsem.at[1])
    pdrain(P - 2, 0)
    pdrain(P - 1, 1)


def kernel(input_ids, table, learned):
    mesh = plsc.VectorSubcoreMesh(core_axis_name="c", subcore_axis_name="s")
    run = functools.partial(
        pl.kernel,
        mesh=mesh,
        out_type=jax.ShapeDtypeStruct((T, DT, NW, 8, LB), jnp.float32),
        scratch_types=[
            pltpu.VMEM((S, LB), jnp.int32),
            pltpu.VMEM((P, D), jnp.float32),
            pltpu.VMEM((NBUF, LB, D), jnp.float32),
            pltpu.VMEM((2, D, LB), jnp.float32),
            pltpu.VMEM((2, D, LB), jnp.float32),
            pltpu.SemaphoreType.DMA((NBUF,)),
            pltpu.SemaphoreType.DMA((2,)),
            pltpu.SemaphoreType.DMA((2,)),
        ],
        compiler_params=pltpu.CompilerParams(
            use_tc_tiling_on_sc=False, needs_layout_passes=False
        ),
    )(_soft_embed)
    x = run(input_ids.T, table, learned)
    return x.transpose(2, 4, 0, 1, 3).reshape(B, T, D)


# speedup vs baseline: 1.5271x; 1.5271x over previous
import functools

import jax
import jax.numpy as jnp
from jax import lax
from jax.experimental import pallas as pl
from jax.experimental.pallas import tpu as pltpu
from jax.experimental.pallas import tpu_sc as plsc

B = 4096
S = 50
D = 64
P = 20
T = P + S

NC = 2
NS = 16
NW = NC * NS
L = 16
DT = D // 8
LB = 128
NBUF = 4


def _splat(val):
    return jnp.full((L,), val, dtype=jnp.int32)


def _soft_embed(ids_hbm, table_hbm, learned_hbm, out_hbm,
                idx_v, lrn_v, rows_v, slab_v, pslab_v, gsem, wsem, psem):
    wid = lax.axis_index("s") * NC + lax.axis_index("c")
    pltpu.sync_copy(ids_hbm.at[:, pl.ds(wid * LB, LB)], idx_v)
    pltpu.sync_copy(learned_hbm, lrn_v)

    def fire(s, p4):
        pltpu.async_copy(table_hbm.at[idx_v.at[s]], rows_v.at[p4], gsem.at[p4])

    def drain_gather(s, p4):
        pltpu.make_async_copy(
            table_hbm.at[idx_v.at[s]], rows_v.at[p4], gsem.at[p4]
        ).wait()

    def write_slab(t, p, sem):
        for dt in range(DT):
            pltpu.async_copy(
                slab_v.at[p, pl.ds(dt * 8, 8)], out_hbm.at[t, dt, wid], sem
            )

    def drain_slab(t, p, sem):
        for dt in range(DT):
            pltpu.make_async_copy(
                slab_v.at[p, pl.ds(dt * 8, 8)], out_hbm.at[t, dt, wid], sem
            ).wait()

    for s0 in range(NBUF):
        fire(s0, s0)

    iota = lax.iota(jnp.int32, L)

    def pdrain(t, p2):
        for dt in range(DT):
            pltpu.make_async_copy(
                pslab_v.at[p2, pl.ds(dt * 8, 8)], out_hbm.at[t, dt, wid],
                psem.at[p2],
            ).wait()

    def prefix_body(t, carry):
        p2 = lax.rem(t, 2)
        p2v = _splat(p2)
        tv = _splat(t)

        @pl.when(t >= 2)
        def _():
            pdrain(t - 2, p2)

        def dbody(d, c):
            v = plsc.load_gather(lrn_v, [tv, _splat(d)])
            for c8 in range(LB // L):
                pslab_v[p2, d, pl.ds(c8 * L, L)] = v
            return c

        lax.fori_loop(0, D, dbody, 0)
        for dt in range(DT):
            pltpu.async_copy(
                pslab_v.at[p2, pl.ds(dt * 8, 8)], out_hbm.at[t, dt, wid],
                psem.at[p2],
            )
        return carry

    lax.fori_loop(0, P, prefix_body, 0)

    def body(s, carry):
        p = lax.rem(s, 2)
        p4 = lax.rem(s, NBUF)
        pv = _splat(p)
        p4v = _splat(p4)
        drain_gather(s, p4)

        @pl.when(s >= 2)
        def _():
            drain_slab(P + s - 2, p, wsem.at[p])

        @plsc.parallel_loop(0, L, 1, unroll=2)
        def kbody(k):
            rot = jnp.bitwise_and(iota + k, L - 1)
            for d0 in range(0, D, L):
                colv = rot + d0
                for bl0 in range(0, LB, L):
                    rowv = iota + bl0
                    v = plsc.load_gather(rows_v, [p4v, rowv, colv])
                    plsc.store_scatter(slab_v, [pv, colv, rowv], v)

        @pl.when(s < S - NBUF)
        def _():
            fire(s + NBUF, p4)

        write_slab(P + s, p, wsem.at[p])
        return carry

    lax.fori_loop(0, S, body, 0)
    drain_slab(P + S - 2, 0, wsem.at[0])
    drain_slab(P + S - 1, 1, wsem.at[1])
    pdrain(P - 2, 0)
    pdrain(P - 1, 1)


def kernel(input_ids, table, learned):
    mesh = plsc.VectorSubcoreMesh(core_axis_name="c", subcore_axis_name="s")
    run = functools.partial(
        pl.kernel,
        mesh=mesh,
        out_type=jax.ShapeDtypeStruct((T, DT, NW, 8, LB), jnp.float32),
        scratch_types=[
            pltpu.VMEM((S, LB), jnp.int32),
            pltpu.VMEM((P, D), jnp.float32),
            pltpu.VMEM((NBUF, LB, D), jnp.float32),
            pltpu.VMEM((2, D, LB), jnp.float32),
            pltpu.VMEM((2, D, LB), jnp.float32),
            pltpu.SemaphoreType.DMA((NBUF,)),
            pltpu.SemaphoreType.DMA((2,)),
            pltpu.SemaphoreType.DMA((2,)),
        ],
        compiler_params=pltpu.CompilerParams(
            use_tc_tiling_on_sc=False, needs_layout_passes=False
        ),
    )(_soft_embed)
    x = run(input_ids.T, table, learned)
    return x.transpose(2, 4, 0, 1, 3).reshape(B, T, D)


# speedup vs baseline: 1.5766x; 1.0324x over previous
import functools

import jax
import jax.numpy as jnp
from jax import lax
from jax.experimental import pallas as pl
from jax.experimental.pallas import tpu as pltpu
from jax.experimental.pallas import tpu_sc as plsc

B = 4096
S = 50
D = 64
P = 20
T = P + S

NC = 2
NS = 16
NW = NC * NS
L = 16
DT = D // 8
LB = 128
NBUF = 4


def _splat(val):
    return jnp.full((L,), val, dtype=jnp.int32)


def _soft_embed(ids_hbm, table_hbm, learned_hbm, out_hbm,
                idx_v, lrn_v, rows_v, slab_v, pslab_v, gsem, wsem, psem):
    wid = lax.axis_index("s") * NC + lax.axis_index("c")
    pltpu.sync_copy(ids_hbm.at[:, pl.ds(wid * LB, LB)], idx_v)
    pltpu.sync_copy(learned_hbm, lrn_v)

    def fire(s, p4):
        pltpu.async_copy(table_hbm.at[idx_v.at[s]], rows_v.at[p4], gsem.at[p4])

    def drain_gather(s, p4):
        pltpu.make_async_copy(
            table_hbm.at[idx_v.at[s]], rows_v.at[p4], gsem.at[p4]
        ).wait()

    def write_slab(t, p, sem):
        for dt in range(DT):
            pltpu.async_copy(
                slab_v.at[p, pl.ds(dt * 8, 8)], out_hbm.at[t, dt, wid], sem
            )

    def drain_slab(t, p, sem):
        for dt in range(DT):
            pltpu.make_async_copy(
                slab_v.at[p, pl.ds(dt * 8, 8)], out_hbm.at[t, dt, wid], sem
            ).wait()

    for s0 in range(NBUF):
        fire(s0, s0)

    iota = lax.iota(jnp.int32, L)

    def pdrain(t, p2):
        for dt in range(DT):
            pltpu.make_async_copy(
                pslab_v.at[p2, pl.ds(dt * 8, 8)], out_hbm.at[t, dt, wid],
                psem.at[p2],
            ).wait()

    def prefix_body(t, carry):
        p2 = lax.rem(t, 2)
        p2v = _splat(p2)
        tv = _splat(t)

        @pl.when(t >= 2)
        def _():
            pdrain(t - 2, p2)

        @plsc.parallel_loop(0, D, 1, unroll=4)
        def dbody(d):
            v = plsc.load_gather(lrn_v, [tv, _splat(d)])
            for c8 in range(LB // L):
                pslab_v[p2, d, pl.ds(c8 * L, L)] = v
        for dt in range(DT):
            pltpu.async_copy(
                pslab_v.at[p2, pl.ds(dt * 8, 8)], out_hbm.at[t, dt, wid],
                psem.at[p2],
            )
        return carry

    lax.fori_loop(0, P, prefix_body, 0)

    def body(s, carry):
        p = lax.rem(s, 2)
        p4 = lax.rem(s, NBUF)
        pv = _splat(p)
        p4v = _splat(p4)
        drain_gather(s, p4)

        @pl.when(s >= 2)
        def _():
            drain_slab(P + s - 2, p, wsem.at[p])

        @plsc.parallel_loop(0, L, 1, unroll=4)
        def kbody(k):
            rot = jnp.bitwise_and(iota + k, L - 1)
            for d0 in range(0, D, L):
                colv = rot + d0
                for bl0 in range(0, LB, L):
                    rowv = iota + bl0
                    v = plsc.load_gather(rows_v, [p4v, rowv, colv])
                    plsc.store_scatter(slab_v, [pv, colv, rowv], v)

        @pl.when(s < S - NBUF)
        def _():
            fire(s + NBUF, p4)

        write_slab(P + s, p, wsem.at[p])
        return carry

    lax.fori_loop(0, S, body, 0)
    drain_slab(P + S - 2, 0, wsem.at[0])
    drain_slab(P + S - 1, 1, wsem.at[1])
    pdrain(P - 2, 0)
    pdrain(P - 1, 1)


def kernel(input_ids, table, learned):
    mesh = plsc.VectorSubcoreMesh(core_axis_name="c", subcore_axis_name="s")
    run = functools.partial(
        pl.kernel,
        mesh=mesh,
        out_type=jax.ShapeDtypeStruct((T, DT, NW, 8, LB), jnp.float32),
        scratch_types=[
            pltpu.VMEM((S, LB), jnp.int32),
            pltpu.VMEM((P, D), jnp.float32),
            pltpu.VMEM((NBUF, LB, D), jnp.float32),
            pltpu.VMEM((2, D, LB), jnp.float32),
            pltpu.VMEM((2, D, LB), jnp.float32),
            pltpu.SemaphoreType.DMA((NBUF,)),
            pltpu.SemaphoreType.DMA((2,)),
            pltpu.SemaphoreType.DMA((2,)),
        ],
        compiler_params=pltpu.CompilerParams(
            use_tc_tiling_on_sc=False, needs_layout_passes=False
        ),
    )(_soft_embed)
    x = run(input_ids.T, table, learned)
    return x.transpose(2, 4, 0, 1, 3).reshape(B, T, D)


# speedup vs baseline: 1.5787x; 1.0013x over previous
import functools

import jax
import jax.numpy as jnp
from jax import lax
from jax.experimental import pallas as pl
from jax.experimental.pallas import tpu as pltpu
from jax.experimental.pallas import tpu_sc as plsc

B = 4096
S = 50
D = 64
P = 20
T = P + S

NC = 2
NS = 16
NW = NC * NS
L = 16
DT = D // 8
LB = 128
NBUF = 4


def _splat(val):
    return jnp.full((L,), val, dtype=jnp.int32)


def _soft_embed(ids_hbm, table_hbm, learned_hbm, out_hbm,
                idx_v, lrn_v, rows_v, slab_v, pslab_v, gsem, wsem, psem):
    wid = lax.axis_index("s") * NC + lax.axis_index("c")
    pltpu.sync_copy(ids_hbm.at[:, pl.ds(wid * LB, LB)], idx_v)
    pltpu.sync_copy(learned_hbm, lrn_v)

    def fire(s, p4):
        pltpu.async_copy(table_hbm.at[idx_v.at[s]], rows_v.at[p4], gsem.at[p4])

    def drain_gather(s, p4):
        pltpu.make_async_copy(
            table_hbm.at[idx_v.at[s]], rows_v.at[p4], gsem.at[p4]
        ).wait()

    def write_slab(t, p, sem):
        for dt in range(DT):
            pltpu.async_copy(
                slab_v.at[p, pl.ds(dt * 8, 8)], out_hbm.at[t, dt, wid], sem
            )

    def drain_slab(t, p, sem):
        for dt in range(DT):
            pltpu.make_async_copy(
                slab_v.at[p, pl.ds(dt * 8, 8)], out_hbm.at[t, dt, wid], sem
            ).wait()

    for s0 in range(NBUF):
        fire(s0, s0)

    iota = lax.iota(jnp.int32, L)

    def pdrain(t, p2):
        for dt in range(DT):
            pltpu.make_async_copy(
                pslab_v.at[p2, pl.ds(dt * 8, 8)], out_hbm.at[t, dt, wid],
                psem.at[p2],
            ).wait()

    def prefix_body(t, carry):
        p2 = lax.rem(t, 2)
        p2v = _splat(p2)
        tv = _splat(t)

        @pl.when(t >= 2)
        def _():
            pdrain(t - 2, p2)

        @plsc.parallel_loop(0, D, 1, unroll=4)
        def dbody(d):
            v = plsc.load_gather(lrn_v, [tv, _splat(d)])
            for c8 in range(LB // L):
                pslab_v[p2, d, pl.ds(c8 * L, L)] = v
        for dt in range(DT):
            pltpu.async_copy(
                pslab_v.at[p2, pl.ds(dt * 8, 8)], out_hbm.at[t, dt, wid],
                psem.at[p2],
            )
        return carry

    lax.fori_loop(0, P, prefix_body, 0)

    def body(s, carry):
        p = lax.rem(s, 2)
        p4 = lax.rem(s, NBUF)
        pv = _splat(p)
        p4v = _splat(p4)
        drain_gather(s, p4)

        @pl.when(s >= 2)
        def _():
            drain_slab(P + s - 2, p, wsem.at[p])

        @plsc.parallel_loop(0, L, 1, unroll=8)
        def kbody(k):
            rot = jnp.bitwise_and(iota + k, L - 1)
            for d0 in range(0, D, L):
                colv = rot + d0
                for bl0 in range(0, LB, L):
                    rowv = iota + bl0
                    v = plsc.load_gather(rows_v, [p4v, rowv, colv])
                    plsc.store_scatter(slab_v, [pv, colv, rowv], v)

        @pl.when(s < S - NBUF)
        def _():
            fire(s + NBUF, p4)

        write_slab(P + s, p, wsem.at[p])
        return carry

    lax.fori_loop(0, S, body, 0)
    drain_slab(P + S - 2, 0, wsem.at[0])
    drain_slab(P + S - 1, 1, wsem.at[1])
    pdrain(P - 2, 0)
    pdrain(P - 1, 1)


def kernel(input_ids, table, learned):
    mesh = plsc.VectorSubcoreMesh(core_axis_name="c", subcore_axis_name="s")
    run = functools.partial(
        pl.kernel,
        mesh=mesh,
        out_type=jax.ShapeDtypeStruct((T, DT, NW, 8, LB), jnp.float32),
        scratch_types=[
            pltpu.VMEM((S, LB), jnp.int32),
            pltpu.VMEM((P, D), jnp.float32),
            pltpu.VMEM((NBUF, LB, D), jnp.float32),
            pltpu.VMEM((2, D, LB), jnp.float32),
            pltpu.VMEM((2, D, LB), jnp.float32),
            pltpu.SemaphoreType.DMA((NBUF,)),
            pltpu.SemaphoreType.DMA((2,)),
            pltpu.SemaphoreType.DMA((2,)),
        ],
        compiler_params=pltpu.CompilerParams(
            use_tc_tiling_on_sc=False, needs_layout_passes=False
        ),
    )(_soft_embed)
    x = run(input_ids.T, table, learned)
    return x.transpose(2, 4, 0, 1, 3).reshape(B, T, D)


# speedup vs baseline: 1.5849x; 1.0039x over previous
import functools

import jax
import jax.numpy as jnp
from jax import lax
from jax.experimental import pallas as pl
from jax.experimental.pallas import tpu as pltpu
from jax.experimental.pallas import tpu_sc as plsc

B = 4096
S = 50
D = 64
P = 20
T = P + S

NC = 2
NS = 16
NW = NC * NS
L = 16
DT = D // 8
LB = 128
NBUF = 4


def _splat(val):
    return jnp.full((L,), val, dtype=jnp.int32)


def _soft_embed(ids_hbm, table_hbm, learned_hbm, out_hbm,
                idx_v, lrn_v, rows_v, slab_v, pslab_v, gsem, wsem, psem):
    wid = lax.axis_index("s") * NC + lax.axis_index("c")
    pltpu.sync_copy(ids_hbm.at[:, pl.ds(wid * LB, LB)], idx_v)
    pltpu.sync_copy(learned_hbm, lrn_v)

    def fire(s, p4):
        pltpu.async_copy(table_hbm.at[idx_v.at[s]], rows_v.at[p4], gsem.at[p4])

    def drain_gather(s, p4):
        pltpu.make_async_copy(
            table_hbm.at[idx_v.at[s]], rows_v.at[p4], gsem.at[p4]
        ).wait()

    def write_slab(t, p, sem):
        pltpu.async_copy(slab_v.at[p], out_hbm.at[t, :, wid], sem)

    def drain_slab(t, p, sem):
        pltpu.make_async_copy(
            slab_v.at[p], out_hbm.at[t, :, wid], sem
        ).wait()

    for s0 in range(NBUF):
        fire(s0, s0)

    iota = lax.iota(jnp.int32, L)

    def pdrain(t, p2):
        pltpu.make_async_copy(
            pslab_v.at[p2], out_hbm.at[t, :, wid], psem.at[p2]
        ).wait()

    def prefix_body(t, carry):
        p2 = lax.rem(t, 2)
        p2v = _splat(p2)
        tv = _splat(t)

        @pl.when(t >= 2)
        def _():
            pdrain(t - 2, p2)

        @plsc.parallel_loop(0, D, 1, unroll=4)
        def dbody(d):
            v = plsc.load_gather(lrn_v, [tv, _splat(d)])
            dt = lax.shift_right_logical(d, 3)
            ds = jnp.bitwise_and(d, 7)
            for c8 in range(LB // L):
                pslab_v[p2, dt, ds, pl.ds(c8 * L, L)] = v
        pltpu.async_copy(pslab_v.at[p2], out_hbm.at[t, :, wid], psem.at[p2])
        return carry

    lax.fori_loop(0, P, prefix_body, 0)

    def body(s, carry):
        p = lax.rem(s, 2)
        p4 = lax.rem(s, NBUF)
        pv = _splat(p)
        p4v = _splat(p4)
        drain_gather(s, p4)

        @pl.when(s >= 2)
        def _():
            drain_slab(P + s - 2, p, wsem.at[p])

        @plsc.parallel_loop(0, L, 1, unroll=4)
        def kbody(k):
            rot = jnp.bitwise_and(iota + k, L - 1)
            for d0 in range(0, D, L):
                colv = rot + d0
                for bl0 in range(0, LB, L):
                    rowv = iota + bl0
                    v = plsc.load_gather(rows_v, [p4v, rowv, colv])
                    dtv = lax.shift_right_logical(colv, 3)
                    dsv = jnp.bitwise_and(colv, 7)
                    plsc.store_scatter(slab_v, [pv, dtv, dsv, rowv], v)

        @pl.when(s < S - NBUF)
        def _():
            fire(s + NBUF, p4)

        write_slab(P + s, p, wsem.at[p])
        return carry

    lax.fori_loop(0, S, body, 0)
    drain_slab(P + S - 2, 0, wsem.at[0])
    drain_slab(P + S - 1, 1, wsem.at[1])
    pdrain(P - 2, 0)
    pdrain(P - 1, 1)


def kernel(input_ids, table, learned):
    mesh = plsc.VectorSubcoreMesh(core_axis_name="c", subcore_axis_name="s")
    run = functools.partial(
        pl.kernel,
        mesh=mesh,
        out_type=jax.ShapeDtypeStruct((T, DT, NW, 8, LB), jnp.float32),
        scratch_types=[
            pltpu.VMEM((S, LB), jnp.int32),
            pltpu.VMEM((P, D), jnp.float32),
            pltpu.VMEM((NBUF, LB, D), jnp.float32),
            pltpu.VMEM((2, DT, 8, LB), jnp.float32),
            pltpu.VMEM((2, DT, 8, LB), jnp.float32),
            pltpu.SemaphoreType.DMA((NBUF,)),
            pltpu.SemaphoreType.DMA((2,)),
            pltpu.SemaphoreType.DMA((2,)),
        ],
        compiler_params=pltpu.CompilerParams(
            use_tc_tiling_on_sc=False, needs_layout_passes=False
        ),
    )(_soft_embed)
    x = run(input_ids.T, table, learned)
    return x.transpose(2, 4, 0, 1, 3).reshape(B, T, D)


# speedup vs baseline: 1.5860x; 1.0007x over previous
import functools

import jax
import jax.numpy as jnp
from jax import lax
from jax.experimental import pallas as pl
from jax.experimental.pallas import tpu as pltpu
from jax.experimental.pallas import tpu_sc as plsc

B = 4096
S = 50
D = 64
P = 20
T = P + S

NC = 2
NS = 16
NW = NC * NS
L = 16
DT = D // 8
LB = 128
NBUF = 4


def _splat(val):
    return jnp.full((L,), val, dtype=jnp.int32)


def _soft_embed(ids_hbm, table_hbm, learned_hbm, out_hbm,
                idx_v, lrn_v, rows_v, slab_v, pslab_v, gsem, wsem, psem):
    wid = lax.axis_index("s") * NC + lax.axis_index("c")
    pltpu.sync_copy(ids_hbm.at[:, pl.ds(wid * LB, LB)], idx_v)
    pltpu.sync_copy(learned_hbm, lrn_v)

    def fire(s, p4):
        pltpu.async_copy(table_hbm.at[idx_v.at[s]], rows_v.at[p4], gsem.at[p4])

    def drain_gather(s, p4):
        pltpu.make_async_copy(
            table_hbm.at[idx_v.at[s]], rows_v.at[p4], gsem.at[p4]
        ).wait()

    def write_slab(t, p, sem):
        pltpu.async_copy(slab_v.at[p], out_hbm.at[t, :, wid], sem)

    def drain_slab(t, p, sem):
        pltpu.make_async_copy(
            slab_v.at[p], out_hbm.at[t, :, wid], sem
        ).wait()

    for s0 in range(NBUF):
        fire(s0, s0)

    iota = lax.iota(jnp.int32, L)

    def pdrain(t, p2):
        pltpu.make_async_copy(
            pslab_v.at[p2], out_hbm.at[t, :, wid], psem.at[p2]
        ).wait()

    def prefix_body(t, carry):
        p2 = lax.rem(t, 2)
        tv = _splat(t)

        @pl.when(t >= 2)
        def _():
            pdrain(t - 2, p2)

        @plsc.parallel_loop(0, D, 1, unroll=4)
        def dbody(d):
            v = plsc.load_gather(lrn_v, [tv, _splat(d)])
            dt = lax.shift_right_logical(d, 3)
            ds = jnp.bitwise_and(d, 7)
            for c8 in range(LB // L):
                pslab_v[p2, dt, ds, pl.ds(c8 * L, L)] = v
        pltpu.async_copy(pslab_v.at[p2], out_hbm.at[t, :, wid], psem.at[p2])
        return carry

    lax.fori_loop(0, P, prefix_body, 0)

    def body(s, carry):
        p = lax.rem(s, 2)
        p4 = lax.rem(s, NBUF)
        pv = _splat(p)
        p4v = _splat(p4)
        drain_gather(s, p4)

        @pl.when(s >= 2)
        def _():
            drain_slab(P + s - 2, p, wsem.at[p])

        @plsc.parallel_loop(0, L, 1, unroll=4)
        def kbody(k):
            rot = jnp.bitwise_and(iota + k, L - 1)
            for d0 in range(0, D, L):
                colv = rot + d0
                for bl0 in range(0, LB, L):
                    rowv = iota + bl0
                    v = plsc.load_gather(rows_v, [p4v, rowv, colv])
                    dtv = lax.shift_right_logical(colv, 3)
                    dsv = jnp.bitwise_and(colv, 7)
                    plsc.store_scatter(slab_v, [pv, dtv, dsv, rowv], v)

        @pl.when(s < S - NBUF)
        def _():
            fire(s + NBUF, p4)

        write_slab(P + s, p, wsem.at[p])
        return carry

    lax.fori_loop(0, S, body, 0)
    drain_slab(P + S - 2, 0, wsem.at[0])
    drain_slab(P + S - 1, 1, wsem.at[1])
    pdrain(P - 2, 0)
    pdrain(P - 1, 1)


def kernel(input_ids, table, learned):
    mesh = plsc.VectorSubcoreMesh(core_axis_name="c", subcore_axis_name="s")
    run = functools.partial(
        pl.kernel,
        mesh=mesh,
        out_type=jax.ShapeDtypeStruct((T, DT, NW, 8, LB), jnp.float32),
        scratch_types=[
            pltpu.VMEM((S, LB), jnp.int32),
            pltpu.VMEM((P, D), jnp.float32),
            pltpu.VMEM((NBUF, LB, D), jnp.float32),
            pltpu.VMEM((2, DT, 8, LB), jnp.float32),
            pltpu.VMEM((2, DT, 8, LB), jnp.float32),
            pltpu.SemaphoreType.DMA((NBUF,)),
            pltpu.SemaphoreType.DMA((2,)),
            pltpu.SemaphoreType.DMA((2,)),
        ],
        compiler_params=pltpu.CompilerParams(
            use_tc_tiling_on_sc=False, needs_layout_passes=False
        ),
    )(_soft_embed)
    x = run(input_ids.T, table, learned)
    return x.transpose(2, 4, 0, 1, 3).reshape(B, T, D)
